# Initial kernel scaffold; baseline (speedup 1.0000x reference)
import jax
import jax.numpy as jnp
from jax.experimental import pallas as pl


def kernel(feat, y, idx, memory, W, b):
    raise NotImplementedError("write your pallas kernel here")


# speedup vs baseline: 4.9541x; 4.9541x over previous
import functools

import jax
import jax.numpy as jnp
from jax import lax
from jax.experimental import pallas as pl
from jax.experimental.pallas import tpu as pltpu
from jax.experimental.pallas import tpu_sc as plsc

B = 1024
K1 = 256
D = 128
NLEM = 500000
T = 0.07
MOM = 0.5

NC = 2
NS = 16
NW = NC * NS
GPT = (B * K1) // NW
CHUNK = 128
NCHUNK = GPT // CHUNK
YPT = B // NW


def _a_body(feat_ref, wt_ref, b_ref, ycol_ref, yrow_ref, fproj_ref, li_ref):
    feat = feat_ref[...]
    proj = jnp.dot(feat, wt_ref[...], preferred_element_type=jnp.float32)
    proj = proj + b_ref[...]
    ones = jnp.ones((D, 1), jnp.float32)
    norm2 = lax.dot_general(proj * proj, ones, (((1,), (0,)), ((), ())),
                            preferred_element_type=jnp.float32)
    fproj_ref[...] = proj * lax.rsqrt(norm2)
    eq = ycol_ref[...] == yrow_ref[...]
    jidx = lax.broadcasted_iota(jnp.int32, (B, B), 1)
    li_ref[...] = jnp.max(jnp.where(eq, jidx, -1), axis=1, keepdims=True)


def _run_a(feat, Wt, b2, ycol, yrow):
    return pl.pallas_call(
        _a_body,
        out_shape=(jax.ShapeDtypeStruct((B, D), jnp.float32),
                   jax.ShapeDtypeStruct((B, 1), jnp.int32)),
    )(feat, Wt, b2, ycol, yrow)


def _b_body(idx_hbm, y_hbm, li_hbm, mem_hbm, feat_hbm,
            gath_hbm, pos_hbm,
            idx_v, rows_v, y_v, li_v, yrow_v, frow_v, sem):
    wid = lax.axis_index("s") * NC + lax.axis_index("c")
    base = pl.multiple_of(wid * GPT, 8)

    def chunk(c, _):
        off = pl.multiple_of(base + c * CHUNK, 8)
        pltpu.sync_copy(idx_hbm.at[pl.ds(off, CHUNK)], idx_v)
        pltpu.async_copy(mem_hbm.at[idx_v], rows_v, sem).wait()
        pltpu.sync_copy(rows_v, gath_hbm.at[pl.ds(off, CHUNK)])
        return 0

    lax.fori_loop(0, NCHUNK, chunk, 0, unroll=False)

    ybase = pl.multiple_of(wid * YPT, 8)
    pltpu.sync_copy(y_hbm.at[pl.ds(ybase, YPT)], y_v)
    pltpu.async_copy(mem_hbm.at[y_v], yrow_v, sem).wait()
    pltpu.sync_copy(li_hbm.at[pl.ds(ybase, YPT)], li_v)
    pltpu.async_copy(feat_hbm.at[li_v], frow_v, sem).wait()
    for r in range(YPT):
        for c in range(D // 16):
            s = pl.ds(c * 16, 16)
            yrow_v[r, s] = yrow_v[r, s] * MOM + frow_v[r, s] * (1.0 - MOM)
    pltpu.sync_copy(yrow_v, pos_hbm.at[pl.ds(ybase, YPT)])


def _run_b(idx_flat, y, li_flat, memory, feat):
    mesh = plsc.VectorSubcoreMesh(core_axis_name="c", subcore_axis_name="s")
    fn = pl.kernel(
        _b_body,
        out_type=(jax.ShapeDtypeStruct((B * K1, D), jnp.float32),
                  jax.ShapeDtypeStruct((B, D), jnp.float32)),
        mesh=mesh,
        scratch_types=[
            pltpu.VMEM((CHUNK,), jnp.int32),
            pltpu.VMEM((CHUNK, D), jnp.float32),
            pltpu.VMEM((YPT,), jnp.int32),
            pltpu.VMEM((YPT,), jnp.int32),
            pltpu.VMEM((YPT, D), jnp.float32),
            pltpu.VMEM((YPT, D), jnp.float32),
            pltpu.SemaphoreType.DMA,
        ],
    )
    return fn(idx_flat, y, li_flat, memory, feat)


def _c_body(g_ref, f_ref, wt_ref, b_ref, out_ref):
    rows = g_ref[...]
    proj = jnp.dot(rows, wt_ref[...], preferred_element_type=jnp.float32)
    proj = proj + b_ref[...]
    f = f_ref[...].reshape(1, D)
    num = lax.dot_general(f, proj, (((1,), (1,)), ((), ())),
                          preferred_element_type=jnp.float32)
    ones = jnp.ones((1, D), jnp.float32)
    norm2 = lax.dot_general(ones, proj * proj, (((1,), (1,)), ((), ())),
                            preferred_element_type=jnp.float32)
    out_ref[...] = (num * lax.rsqrt(norm2) * (1.0 / T)).reshape(1, 1, K1)


def _run_c(gathered, fproj, Wt, b2):
    out = pl.pallas_call(
        _c_body,
        grid=(B,),
        in_specs=[
            pl.BlockSpec((K1, D), lambda i: (i, 0)),
            pl.BlockSpec((1, 1, D), lambda i: (i, 0, 0)),
            pl.BlockSpec((D, D), lambda i: (0, 0)),
            pl.BlockSpec((1, D), lambda i: (0, 0)),
        ],
        out_specs=pl.BlockSpec((1, 1, K1), lambda i: (i, 0, 0)),
        out_shape=jax.ShapeDtypeStruct((B, 1, K1), jnp.float32),
    )(gathered.reshape(B * K1, D), fproj.reshape(B, 1, D), Wt, b2)
    return out.reshape(B, K1)


def _e_body(mem_ref, y_ref, pos_ref, out_ref, sem):
    del mem_ref

    def issue(i, _):
        r = y_ref[i]
        pltpu.make_async_copy(pos_ref.at[pl.ds(i, 1)],
                              out_ref.at[pl.ds(r, 1)], sem).start()
        return 0

    lax.fori_loop(0, B, issue, 0)

    def drain(i, _):
        pltpu.make_async_copy(pos_ref.at[pl.ds(0, 1)],
                              out_ref.at[pl.ds(0, 1)], sem).wait()
        return 0

    lax.fori_loop(0, B, drain, 0)


def _run_e(memory, y, pos):
    return pl.pallas_call(
        _e_body,
        in_specs=[
            pl.BlockSpec(memory_space=pl.ANY),
            pl.BlockSpec(memory_space=pltpu.SMEM),
            pl.BlockSpec(memory_space=pltpu.VMEM),
        ],
        out_specs=pl.BlockSpec(memory_space=pl.ANY),
        out_shape=jax.ShapeDtypeStruct((NLEM, D), jnp.float32),
        input_output_aliases={0: 0},
        scratch_shapes=[pltpu.SemaphoreType.DMA],
    )(memory, y, pos)


def kernel(feat, y, idx, memory, W, b):
    Wt = W.T
    b2 = b.reshape(1, D)
    y = y.astype(jnp.int32)
    ycol = y.reshape(B, 1)
    yrow = y.reshape(1, B)
    fproj, li = _run_a(feat, Wt, b2, ycol, yrow)
    idx_flat = idx.reshape(-1).astype(jnp.int32)
    gathered, pos = _run_b(idx_flat, y, li.reshape(B), memory, feat)
    out = _run_c(gathered, fproj, Wt, b2)
    new_memory = _run_e(memory, y, pos)
    return (out.reshape(B, K1, 1), new_memory)
